# Initial kernel scaffold; baseline (speedup 1.0000x reference)
import jax
import jax.numpy as jnp
from jax.experimental import pallas as pl


def kernel(x, edge_index, edge_attr, params):
    raise NotImplementedError("write your pallas kernel here")


# speedup vs baseline: 1.2089x; 1.2089x over previous
import functools

import jax
import jax.numpy as jnp
from jax import lax
from jax.experimental import pallas as pl
from jax.experimental.pallas import tpu as pltpu
from jax.experimental.pallas import tpu_sc as plsc

NC = 2
NS = 16
NW = NC * NS
LANES = 16
CH = 128

_PREC = jax.lax.Precision.HIGHEST


def _mesh():
    return plsc.VectorSubcoreMesh(core_axis_name="c", subcore_axis_name="s")


def _sc_gather2(x, row, col):
    n, h = x.shape
    e = row.shape[0]
    assert e % NW == 0
    epw = e // NW
    n_full, rem = divmod(epw, CH)
    assert epw % 8 == 0 and rem % 8 == 0

    out_t = jax.ShapeDtypeStruct((e, h), x.dtype)
    scratch = [
        pltpu.VMEM((CH,), jnp.int32), pltpu.VMEM((CH,), jnp.int32),
        pltpu.VMEM((CH, h), x.dtype), pltpu.VMEM((CH, h), x.dtype),
    ]
    if rem:
        scratch += [
            pltpu.VMEM((rem,), jnp.int32), pltpu.VMEM((rem,), jnp.int32),
            pltpu.VMEM((rem, h), x.dtype), pltpu.VMEM((rem, h), x.dtype),
        ]

    @functools.partial(pl.kernel, out_type=(out_t, out_t), mesh=_mesh(),
                       scratch_types=scratch)
    def k(x_hbm, row_hbm, col_hbm, src_hbm, dst_hbm,
          idxr, idxc, bufr, bufc, *tail):
        wid = lax.axis_index("s") * NC + lax.axis_index("c")
        base_w = wid * epw

        @pl.loop(0, n_full)
        def _(i):
            b = base_w + i * CH
            pltpu.sync_copy(row_hbm.at[pl.ds(b, CH)], idxr)
            pltpu.sync_copy(col_hbm.at[pl.ds(b, CH)], idxc)
            pltpu.sync_copy(x_hbm.at[idxr], bufr)
            pltpu.sync_copy(x_hbm.at[idxc], bufc)
            pltpu.sync_copy(bufr, src_hbm.at[pl.ds(b, CH)])
            pltpu.sync_copy(bufc, dst_hbm.at[pl.ds(b, CH)])

        if rem:
            idxrt, idxct, bufrt, bufct = tail
            bt = base_w + n_full * CH
            pltpu.sync_copy(row_hbm.at[pl.ds(bt, rem)], idxrt)
            pltpu.sync_copy(col_hbm.at[pl.ds(bt, rem)], idxct)
            pltpu.sync_copy(x_hbm.at[idxrt], bufrt)
            pltpu.sync_copy(x_hbm.at[idxct], bufct)
            pltpu.sync_copy(bufrt, src_hbm.at[pl.ds(bt, rem)])
            pltpu.sync_copy(bufct, dst_hbm.at[pl.ds(bt, rem)])

    return k(x, row, col)


def _sc_segsum(vals, col, zeros):
    e, w = vals.shape
    n = zeros.shape[0]
    assert e % NC == 0
    epc = e // NC
    assert epc % NS == 0
    eps = epc // NS
    n_full, rem = divmod(eps, CH)
    assert eps % 8 == 0 and rem % 8 == 0
    rps = (n // NS) & ~7
    rps_last = n - (NS - 1) * rps
    assert rps % 8 == 0 and rps_last % 8 == 0 and rps_last >= rps

    out_t = jax.ShapeDtypeStruct((NC * n, w), vals.dtype)
    scratch = [
        pltpu.VMEM_SHARED((n, w), vals.dtype),
        pltpu.VMEM((CH,), jnp.int32),
        pltpu.VMEM((CH, w), vals.dtype),
    ]
    if rem:
        scratch += [pltpu.VMEM((rem,), jnp.int32),
                    pltpu.VMEM((rem, w), vals.dtype)]

    @functools.partial(pl.kernel, out_type=out_t, mesh=_mesh(),
                       scratch_types=scratch)
    def k(vals_hbm, col_hbm, zeros_hbm, out_hbm, acc_sh, idx_v, buf_v, *tail):
        core = lax.axis_index("c")
        sid = lax.axis_index("s")
        r0 = sid * rps

        @pl.when(sid < NS - 1)
        def _():
            pltpu.sync_copy(zeros_hbm.at[pl.ds(r0, rps)],
                            acc_sh.at[pl.ds(r0, rps)])

        @pl.when(sid == NS - 1)
        def _():
            pltpu.sync_copy(zeros_hbm.at[pl.ds(r0, rps_last)],
                            acc_sh.at[pl.ds(r0, rps_last)])

        plsc.subcore_barrier()

        base = core * epc + sid * eps

        @pl.loop(0, n_full)
        def _(i):
            b = base + i * CH
            pltpu.sync_copy(col_hbm.at[pl.ds(b, CH)], idx_v)
            pltpu.sync_copy(vals_hbm.at[pl.ds(b, CH)], buf_v)
            pltpu.sync_copy(buf_v, acc_sh.at[idx_v], add=True)

        if rem:
            idx_t, buf_t = tail
            bt = base + n_full * CH
            pltpu.sync_copy(col_hbm.at[pl.ds(bt, rem)], idx_t)
            pltpu.sync_copy(vals_hbm.at[pl.ds(bt, rem)], buf_t)
            pltpu.sync_copy(buf_t, acc_sh.at[idx_t], add=True)

        plsc.subcore_barrier()

        @pl.when(sid < NS - 1)
        def _():
            pltpu.sync_copy(acc_sh.at[pl.ds(r0, rps)],
                            out_hbm.at[pl.ds(core * n + r0, rps)])

        @pl.when(sid == NS - 1)
        def _():
            pltpu.sync_copy(acc_sh.at[pl.ds(r0, rps_last)],
                            out_hbm.at[pl.ds(core * n + r0, rps_last)])

    return k(vals, col, zeros)


def _dot(a, b):
    return jax.lax.dot_general(a, b, (((1,), (0,)), ((), ())),
                               precision=_PREC,
                               preferred_element_type=jnp.float32)


def _tc_edge(src, dst, ea, pe, pn1):
    e, h = ea.shape
    be = 2560
    assert e % be == 0
    grid = (e // be,)

    w1s, w1d, w1e = pe["W1"][:h], pe["W1"][h:2 * h], pe["W1"][2 * h:]
    v1s, v1e = pn1["W1"][:h], pn1["W1"][h:]
    b1 = pe["b1"].reshape(1, h)
    b2 = pe["b2"].reshape(1, h)
    c1 = pn1["b1"].reshape(1, h)
    c2 = pn1["b2"].reshape(1, h)

    row_spec = pl.BlockSpec((be, h), lambda i: (i, 0))
    w_spec = pl.BlockSpec((h, h), lambda i: (0, 0))
    b_spec = pl.BlockSpec((1, h), lambda i: (0, 0))

    def body(src_r, dst_r, ea_r, w1s_r, w1d_r, w1e_r, b1_r, w2_r, b2_r,
             v1s_r, v1e_r, c1_r, v2_r, c2_r, e_out, m_out):
        s = src_r[...]
        hh = jnp.maximum(
            _dot(s, w1s_r[...]) + _dot(dst_r[...], w1d_r[...])
            + _dot(ea_r[...], w1e_r[...]) + b1_r[...], 0.0)
        e2 = _dot(hh, w2_r[...]) + b2_r[...]
        e_out[...] = e2
        g = jnp.maximum(_dot(s, v1s_r[...]) + _dot(e2, v1e_r[...])
                        + c1_r[...], 0.0)
        m_out[...] = _dot(g, v2_r[...]) + c2_r[...]

    sds = jax.ShapeDtypeStruct((e, h), jnp.float32)
    return pl.pallas_call(
        body,
        grid=grid,
        in_specs=[row_spec, row_spec, row_spec,
                  w_spec, w_spec, w_spec, b_spec, w_spec, b_spec,
                  w_spec, w_spec, b_spec, w_spec, b_spec],
        out_specs=[row_spec, row_spec],
        out_shape=(sds, sds),
        compiler_params=pltpu.CompilerParams(
            dimension_semantics=("parallel",)),
    )(src, dst, ea, w1s, w1d, w1e, b1, pe["W2"], b2,
      v1s, v1e, c1, pn1["W2"], c2)


def _tc_node(x, s0, s1, c0, c1, pn2):
    n, h = x.shape
    bn = 1000
    assert n % bn == 0
    grid = (n // bn,)

    w1x, w1a = pn2["W1"][:h], pn2["W1"][h:]
    b1 = pn2["b1"].reshape(1, h)
    b2 = pn2["b2"].reshape(1, h)

    row_spec = pl.BlockSpec((bn, h), lambda i: (i, 0))
    cnt_spec = pl.BlockSpec((bn, LANES), lambda i: (i, 0))
    w_spec = pl.BlockSpec((h, h), lambda i: (0, 0))
    b_spec = pl.BlockSpec((1, h), lambda i: (0, 0))

    def body(x_r, s0_r, s1_r, c0_r, c1_r, w1x_r, w1a_r, b1_r, w2_r, b2_r,
             out_r):
        cnt = jnp.maximum(c0_r[:, 0:1] + c1_r[:, 0:1], 1.0)
        agg = (s0_r[...] + s1_r[...]) / cnt
        hh = jnp.maximum(
            _dot(x_r[...], w1x_r[...]) + _dot(agg, w1a_r[...]) + b1_r[...],
            0.0)
        out_r[...] = _dot(hh, w2_r[...]) + b2_r[...]

    return pl.pallas_call(
        body,
        grid=grid,
        in_specs=[row_spec, row_spec, row_spec, cnt_spec, cnt_spec,
                  w_spec, w_spec, b_spec, w_spec, b_spec],
        out_specs=pl.BlockSpec((bn, h), lambda i: (i, 0)),
        out_shape=jax.ShapeDtypeStruct((n, h), jnp.float32),
        compiler_params=pltpu.CompilerParams(
            dimension_semantics=("parallel",)),
    )(x, s0, s1, c0, c1, w1x, w1a, b1, pn2["W2"], b2)


def kernel(x, edge_index, edge_attr, params):
    n, h = x.shape
    e = edge_attr.shape[0]
    row = edge_index[0]
    col = edge_index[1]

    ones_e = jnp.ones((e, LANES), jnp.float32)
    zeros_c = jnp.zeros((n, LANES), jnp.float32)
    zeros_s = jnp.zeros((n, h), jnp.float32)
    cnt_parts = _sc_segsum(ones_e, col, zeros_c)
    c0, c1 = cnt_parts[:n], cnt_parts[n:]

    for p in params:
        src, dst = _sc_gather2(x, row, col)
        e_new, m = _tc_edge(src, dst, edge_attr, p["edge"], p["node1"])
        s_parts = _sc_segsum(m, col, zeros_s)
        x = _tc_node(x, s_parts[:n], s_parts[n:], c0, c1, p["node2"])
        edge_attr = e_new
    return (x, edge_attr)


# speedup vs baseline: 1.2278x; 1.0156x over previous
import functools

import jax
import jax.numpy as jnp
from jax import lax
from jax.experimental import pallas as pl
from jax.experimental.pallas import tpu as pltpu
from jax.experimental.pallas import tpu_sc as plsc

NC = 2
NS = 16
NW = NC * NS
LANES = 16
CH = 128

def _mesh():
    return plsc.VectorSubcoreMesh(core_axis_name="c", subcore_axis_name="s")


def _sc_gather2(x, row, col):
    n, h = x.shape
    e = row.shape[0]
    assert e % NW == 0
    epw = e // NW
    n_full, rem = divmod(epw, CH)
    assert epw % 8 == 0 and rem % 8 == 0

    out_t = jax.ShapeDtypeStruct((e, h), x.dtype)
    scratch = [
        pltpu.VMEM((CH,), jnp.int32), pltpu.VMEM((CH,), jnp.int32),
        pltpu.VMEM((CH, h), x.dtype), pltpu.VMEM((CH, h), x.dtype),
    ]
    if rem:
        scratch += [
            pltpu.VMEM((rem,), jnp.int32), pltpu.VMEM((rem,), jnp.int32),
            pltpu.VMEM((rem, h), x.dtype), pltpu.VMEM((rem, h), x.dtype),
        ]

    @functools.partial(pl.kernel, out_type=(out_t, out_t), mesh=_mesh(),
                       scratch_types=scratch)
    def k(x_hbm, row_hbm, col_hbm, src_hbm, dst_hbm,
          idxr, idxc, bufr, bufc, *tail):
        wid = lax.axis_index("s") * NC + lax.axis_index("c")
        base_w = wid * epw

        @pl.loop(0, n_full)
        def _(i):
            b = base_w + i * CH
            pltpu.sync_copy(row_hbm.at[pl.ds(b, CH)], idxr)
            pltpu.sync_copy(col_hbm.at[pl.ds(b, CH)], idxc)
            pltpu.sync_copy(x_hbm.at[idxr], bufr)
            pltpu.sync_copy(x_hbm.at[idxc], bufc)
            pltpu.sync_copy(bufr, src_hbm.at[pl.ds(b, CH)])
            pltpu.sync_copy(bufc, dst_hbm.at[pl.ds(b, CH)])

        if rem:
            idxrt, idxct, bufrt, bufct = tail
            bt = base_w + n_full * CH
            pltpu.sync_copy(row_hbm.at[pl.ds(bt, rem)], idxrt)
            pltpu.sync_copy(col_hbm.at[pl.ds(bt, rem)], idxct)
            pltpu.sync_copy(x_hbm.at[idxrt], bufrt)
            pltpu.sync_copy(x_hbm.at[idxct], bufct)
            pltpu.sync_copy(bufrt, src_hbm.at[pl.ds(bt, rem)])
            pltpu.sync_copy(bufct, dst_hbm.at[pl.ds(bt, rem)])

    return k(x, row, col)


def _sc_segsum(vals, col, zeros, count_mode=False):
    e, w = vals.shape
    n = zeros.shape[0]
    assert e % NC == 0
    epc = e // NC
    assert epc % NS == 0
    eps = epc // NS
    n_full, rem = divmod(eps, CH)
    assert eps % 8 == 0 and rem % 8 == 0
    rps = (n // NS) & ~7
    rps_last = n - (NS - 1) * rps
    assert rps % 8 == 0 and rps_last % 8 == 0 and rps_last >= rps

    out_t = jax.ShapeDtypeStruct((NC * n, w), vals.dtype)
    scratch = [
        pltpu.VMEM_SHARED((n, w), vals.dtype),
        pltpu.VMEM((CH,), jnp.int32),
        pltpu.VMEM((CH, w), vals.dtype),
    ]
    if rem:
        scratch += [pltpu.VMEM((rem,), jnp.int32),
                    pltpu.VMEM((rem, w), vals.dtype)]

    @functools.partial(pl.kernel, out_type=out_t, mesh=_mesh(),
                       scratch_types=scratch)
    def k(vals_hbm, col_hbm, zeros_hbm, out_hbm, acc_sh, idx_v, buf_v, *tail):
        core = lax.axis_index("c")
        sid = lax.axis_index("s")

        if count_mode:
            @pl.loop(0, CH)
            def _(r):
                @pl.loop(0, w // LANES)
                def _(cc):
                    buf_v[r, pl.ds(cc * LANES, LANES)] = jnp.ones(
                        (LANES,), vals.dtype)

        r0 = sid * rps

        @pl.when(sid < NS - 1)
        def _():
            pltpu.sync_copy(zeros_hbm.at[pl.ds(r0, rps)],
                            acc_sh.at[pl.ds(r0, rps)])

        @pl.when(sid == NS - 1)
        def _():
            pltpu.sync_copy(zeros_hbm.at[pl.ds(r0, rps_last)],
                            acc_sh.at[pl.ds(r0, rps_last)])

        plsc.subcore_barrier()

        base = core * epc + sid * eps

        @pl.loop(0, n_full)
        def _(i):
            b = base + i * CH
            pltpu.sync_copy(col_hbm.at[pl.ds(b, CH)], idx_v)
            if not count_mode:
                pltpu.sync_copy(vals_hbm.at[pl.ds(b, CH)], buf_v)
            pltpu.sync_copy(buf_v, acc_sh.at[idx_v], add=True)

        if rem:
            idx_t, buf_t = tail
            bt = base + n_full * CH
            pltpu.sync_copy(col_hbm.at[pl.ds(bt, rem)], idx_t)
            if count_mode:
                @pl.loop(0, rem)
                def _(r):
                    @pl.loop(0, w // LANES)
                    def _(cc):
                        buf_t[r, pl.ds(cc * LANES, LANES)] = jnp.ones(
                            (LANES,), vals.dtype)
            else:
                pltpu.sync_copy(vals_hbm.at[pl.ds(bt, rem)], buf_t)
            pltpu.sync_copy(buf_t, acc_sh.at[idx_t], add=True)

        plsc.subcore_barrier()

        @pl.when(sid < NS - 1)
        def _():
            pltpu.sync_copy(acc_sh.at[pl.ds(r0, rps)],
                            out_hbm.at[pl.ds(core * n + r0, rps)])

        @pl.when(sid == NS - 1)
        def _():
            pltpu.sync_copy(acc_sh.at[pl.ds(r0, rps_last)],
                            out_hbm.at[pl.ds(core * n + r0, rps_last)])

    return k(vals, col, zeros)


def _dot(a, b):
    return jax.lax.dot_general(a, b, (((1,), (0,)), ((), ())),
                               precision=jax.lax.Precision.HIGHEST,
                               preferred_element_type=jnp.float32)


def _tc_edge(src, dst, ea, pe, pn1):
    e, h = ea.shape
    be = 2560
    assert e % be == 0
    grid = (e // be,)

    w1s, w1d, w1e = pe["W1"][:h], pe["W1"][h:2 * h], pe["W1"][2 * h:]
    v1s, v1e = pn1["W1"][:h], pn1["W1"][h:]
    b1 = pe["b1"].reshape(1, h)
    b2 = pe["b2"].reshape(1, h)
    c2 = pn1["b2"].reshape(1, h)
    z = jnp.zeros((h, h), jnp.float32)
    wa = jnp.concatenate([
        jnp.concatenate([w1s, v1s], axis=1),
        jnp.concatenate([w1d, z], axis=1),
        jnp.concatenate([w1e, z], axis=1)], axis=0)
    w2v = jnp.matmul(pe["W2"], v1e, precision=jax.lax.Precision.HIGHEST)
    wb = jnp.concatenate([pe["W2"], w2v], axis=1)
    c1p = (pn1["b1"]
           + jnp.matmul(b2, v1e,
                        precision=jax.lax.Precision.HIGHEST)).reshape(1, h)

    row_spec = pl.BlockSpec((be, h), lambda i: (i, 0))
    wa_spec = pl.BlockSpec((3 * h, 2 * h), lambda i: (0, 0))
    wb_spec = pl.BlockSpec((h, 2 * h), lambda i: (0, 0))
    w_spec = pl.BlockSpec((h, h), lambda i: (0, 0))
    b_spec = pl.BlockSpec((1, h), lambda i: (0, 0))

    def body(src_r, dst_r, ea_r, wa_r, b1_r, wb_r, b2_r, c1p_r, v2_r, c2_r,
             e_out, m_out):
        cat = jnp.concatenate([src_r[...], dst_r[...], ea_r[...]], axis=1)
        a = _dot(cat, wa_r[...])
        hh = jnp.maximum(a[:, :h] + b1_r[...], 0.0)
        gs = a[:, h:]
        bb = _dot(hh, wb_r[...])
        e2 = bb[:, :h] + b2_r[...]
        e_out[...] = e2
        g = jnp.maximum(gs + bb[:, h:] + c1p_r[...], 0.0)
        m_out[...] = _dot(g, v2_r[...]) + c2_r[...]

    sds = jax.ShapeDtypeStruct((e, h), jnp.float32)
    return pl.pallas_call(
        body,
        grid=grid,
        in_specs=[row_spec, row_spec, row_spec,
                  wa_spec, b_spec, wb_spec, b_spec, b_spec, w_spec, b_spec],
        out_specs=[row_spec, row_spec],
        out_shape=(sds, sds),
        compiler_params=pltpu.CompilerParams(
            dimension_semantics=("parallel",)),
    )(src, dst, ea, wa, b1, wb, b2, c1p, pn1["W2"], c2)


def _tc_node(x, s0, s1, c0, c1, pn2):
    n, h = x.shape
    bn = 1000
    assert n % bn == 0
    grid = (n // bn,)

    b1 = pn2["b1"].reshape(1, h)
    b2 = pn2["b2"].reshape(1, h)

    row_spec = pl.BlockSpec((bn, h), lambda i: (i, 0))
    cnt_spec = pl.BlockSpec((bn, h), lambda i: (i, 0))
    w1_spec = pl.BlockSpec((2 * h, h), lambda i: (0, 0))
    w_spec = pl.BlockSpec((h, h), lambda i: (0, 0))
    b_spec = pl.BlockSpec((1, h), lambda i: (0, 0))

    def body(x_r, s0_r, s1_r, c0_r, c1_r, w1_r, b1_r, w2_r, b2_r, out_r):
        cnt = jnp.maximum(c0_r[:, 0:1] + c1_r[:, 0:1], 1.0)
        agg = (s0_r[...] + s1_r[...]) / cnt
        cat = jnp.concatenate([x_r[...], agg], axis=1)
        hh = jnp.maximum(_dot(cat, w1_r[...]) + b1_r[...], 0.0)
        out_r[...] = _dot(hh, w2_r[...]) + b2_r[...]

    return pl.pallas_call(
        body,
        grid=grid,
        in_specs=[row_spec, row_spec, row_spec, cnt_spec, cnt_spec,
                  w1_spec, b_spec, w_spec, b_spec],
        out_specs=pl.BlockSpec((bn, h), lambda i: (i, 0)),
        out_shape=jax.ShapeDtypeStruct((n, h), jnp.float32),
        compiler_params=pltpu.CompilerParams(
            dimension_semantics=("parallel",)),
    )(x, s0, s1, c0, c1, pn2["W1"], b1, pn2["W2"], b2)


def kernel(x, edge_index, edge_attr, params):
    n, h = x.shape
    e = edge_attr.shape[0]
    row = edge_index[0]
    col = edge_index[1]

    zeros_s = jnp.zeros((n, h), jnp.float32)
    cnt_parts = _sc_segsum(edge_attr, col, zeros_s, count_mode=True)
    c0, c1 = cnt_parts[:n], cnt_parts[n:]

    for p in params:
        src, dst = _sc_gather2(x, row, col)
        e_new, m = _tc_edge(src, dst, edge_attr, p["edge"], p["node1"])
        s_parts = _sc_segsum(m, col, zeros_s)
        x = _tc_node(x, s_parts[:n], s_parts[n:], c0, c1, p["node2"])
        edge_attr = e_new
    return (x, edge_attr)


# speedup vs baseline: 2.8351x; 2.3091x over previous
import functools

import jax
import jax.numpy as jnp
from jax import lax
from jax.experimental import pallas as pl
from jax.experimental.pallas import tpu as pltpu
from jax.experimental.pallas import tpu_sc as plsc

NC = 2
NS = 16
NW = NC * NS
LANES = 16
CH = 128

def _mesh():
    return plsc.VectorSubcoreMesh(core_axis_name="c", subcore_axis_name="s")


def _sc_gather2(x, row, col):
    n, h = x.shape
    e = row.shape[0]
    assert e % NW == 0
    epw = e // NW
    n_full, rem = divmod(epw, CH)
    assert epw % 8 == 0 and rem % 8 == 0

    out_t = jax.ShapeDtypeStruct((e, h), x.dtype)
    scratch = [
        pltpu.VMEM((CH,), jnp.int32), pltpu.VMEM((CH,), jnp.int32),
        pltpu.VMEM((CH, h), x.dtype), pltpu.VMEM((CH, h), x.dtype),
    ]
    if rem:
        scratch += [
            pltpu.VMEM((rem,), jnp.int32), pltpu.VMEM((rem,), jnp.int32),
            pltpu.VMEM((rem, h), x.dtype), pltpu.VMEM((rem, h), x.dtype),
        ]

    @functools.partial(pl.kernel, out_type=(out_t, out_t), mesh=_mesh(),
                       scratch_types=scratch)
    def k(x_hbm, row_hbm, col_hbm, src_hbm, dst_hbm,
          idxr, idxc, bufr, bufc, *tail):
        wid = lax.axis_index("s") * NC + lax.axis_index("c")
        base_w = wid * epw

        @pl.loop(0, n_full)
        def _(i):
            b = base_w + i * CH
            pltpu.sync_copy(row_hbm.at[pl.ds(b, CH)], idxr)
            pltpu.sync_copy(col_hbm.at[pl.ds(b, CH)], idxc)
            pltpu.sync_copy(x_hbm.at[idxr], bufr)
            pltpu.sync_copy(x_hbm.at[idxc], bufc)
            pltpu.sync_copy(bufr, src_hbm.at[pl.ds(b, CH)])
            pltpu.sync_copy(bufc, dst_hbm.at[pl.ds(b, CH)])

        if rem:
            idxrt, idxct, bufrt, bufct = tail
            bt = base_w + n_full * CH
            pltpu.sync_copy(row_hbm.at[pl.ds(bt, rem)], idxrt)
            pltpu.sync_copy(col_hbm.at[pl.ds(bt, rem)], idxct)
            pltpu.sync_copy(x_hbm.at[idxrt], bufrt)
            pltpu.sync_copy(x_hbm.at[idxct], bufct)
            pltpu.sync_copy(bufrt, src_hbm.at[pl.ds(bt, rem)])
            pltpu.sync_copy(bufct, dst_hbm.at[pl.ds(bt, rem)])

    return k(x, row, col)


def _sc_segsum(vals, col, zeros, count_mode=False):
    e, w = vals.shape
    n = zeros.shape[0]
    assert e % NC == 0
    epc = e // NC
    assert epc % NS == 0
    eps = epc // NS
    n_full, rem = divmod(eps, CH)
    assert eps % 8 == 0 and rem % 8 == 0
    rps = (n // NS) & ~7
    rps_last = n - (NS - 1) * rps
    assert rps % 8 == 0 and rps_last % 8 == 0 and rps_last >= rps

    out_t = jax.ShapeDtypeStruct((NC * n, w), vals.dtype)
    scratch = [
        pltpu.VMEM_SHARED((n, w), vals.dtype),
        pltpu.VMEM((CH,), jnp.int32),
        pltpu.VMEM((CH, w), vals.dtype),
    ]
    if rem:
        scratch += [pltpu.VMEM((rem,), jnp.int32),
                    pltpu.VMEM((rem, w), vals.dtype)]

    @functools.partial(pl.kernel, out_type=out_t, mesh=_mesh(),
                       scratch_types=scratch)
    def k(vals_hbm, col_hbm, zeros_hbm, out_hbm, acc_sh, idx_v, buf_v, *tail):
        core = lax.axis_index("c")
        sid = lax.axis_index("s")

        if count_mode:
            @pl.loop(0, CH)
            def _(r):
                @pl.loop(0, w // LANES)
                def _(cc):
                    buf_v[r, pl.ds(cc * LANES, LANES)] = jnp.ones(
                        (LANES,), vals.dtype)

        r0 = sid * rps

        @pl.when(sid < NS - 1)
        def _():
            pltpu.sync_copy(zeros_hbm.at[pl.ds(r0, rps)],
                            acc_sh.at[pl.ds(r0, rps)])

        @pl.when(sid == NS - 1)
        def _():
            pltpu.sync_copy(zeros_hbm.at[pl.ds(r0, rps_last)],
                            acc_sh.at[pl.ds(r0, rps_last)])

        plsc.subcore_barrier()

        base = core * epc + sid * eps

        @pl.loop(0, n_full)
        def _(i):
            b = base + i * CH
            pltpu.sync_copy(col_hbm.at[pl.ds(b, CH)], idx_v)
            if not count_mode:
                pltpu.sync_copy(vals_hbm.at[pl.ds(b, CH)], buf_v)
            pltpu.sync_copy(buf_v, acc_sh.at[idx_v], add=True)

        if rem:
            idx_t, buf_t = tail
            bt = base + n_full * CH
            pltpu.sync_copy(col_hbm.at[pl.ds(bt, rem)], idx_t)
            if count_mode:
                @pl.loop(0, rem)
                def _(r):
                    @pl.loop(0, w // LANES)
                    def _(cc):
                        buf_t[r, pl.ds(cc * LANES, LANES)] = jnp.ones(
                            (LANES,), vals.dtype)
            else:
                pltpu.sync_copy(vals_hbm.at[pl.ds(bt, rem)], buf_t)
            pltpu.sync_copy(buf_t, acc_sh.at[idx_t], add=True)

        plsc.subcore_barrier()

        @pl.when(sid < NS - 1)
        def _():
            pltpu.sync_copy(acc_sh.at[pl.ds(r0, rps)],
                            out_hbm.at[pl.ds(core * n + r0, rps)])

        @pl.when(sid == NS - 1)
        def _():
            pltpu.sync_copy(acc_sh.at[pl.ds(r0, rps_last)],
                            out_hbm.at[pl.ds(core * n + r0, rps_last)])

    return k(vals, col, zeros)


def _dot(a, b):
    return jax.lax.dot_general(a.astype(jnp.bfloat16), b.astype(jnp.bfloat16),
                               (((1,), (0,)), ((), ())),
                               preferred_element_type=jnp.float32)


def _tc_edge(src, dst, ea, pe, pn1):
    e, h = ea.shape
    be = 2560
    assert e % be == 0
    grid = (e // be,)

    w1s, w1d, w1e = pe["W1"][:h], pe["W1"][h:2 * h], pe["W1"][2 * h:]
    v1s, v1e = pn1["W1"][:h], pn1["W1"][h:]
    b1 = pe["b1"].reshape(1, h)
    b2 = pe["b2"].reshape(1, h)
    c2 = pn1["b2"].reshape(1, h)
    z = jnp.zeros((h, h), jnp.float32)
    wa = jnp.concatenate([
        jnp.concatenate([w1s, v1s], axis=1),
        jnp.concatenate([w1d, z], axis=1),
        jnp.concatenate([w1e, z], axis=1)], axis=0)
    w2v = jnp.matmul(pe["W2"], v1e, precision=jax.lax.Precision.HIGHEST)
    wb = jnp.concatenate([pe["W2"], w2v], axis=1)
    c1p = (pn1["b1"]
           + jnp.matmul(b2, v1e,
                        precision=jax.lax.Precision.HIGHEST)).reshape(1, h)

    row_spec = pl.BlockSpec((be, h), lambda i: (i, 0))
    wa_spec = pl.BlockSpec((3 * h, 2 * h), lambda i: (0, 0))
    wb_spec = pl.BlockSpec((h, 2 * h), lambda i: (0, 0))
    w_spec = pl.BlockSpec((h, h), lambda i: (0, 0))
    b_spec = pl.BlockSpec((1, h), lambda i: (0, 0))

    def body(src_r, dst_r, ea_r, wa_r, b1_r, wb_r, b2_r, c1p_r, v2_r, c2_r,
             e_out, m_out):
        cat = jnp.concatenate([src_r[...], dst_r[...], ea_r[...]], axis=1)
        a = _dot(cat, wa_r[...])
        hh = jnp.maximum(a[:, :h] + b1_r[...], 0.0)
        gs = a[:, h:]
        bb = _dot(hh, wb_r[...])
        e2 = bb[:, :h] + b2_r[...]
        e_out[...] = e2
        g = jnp.maximum(gs + bb[:, h:] + c1p_r[...], 0.0)
        m_out[...] = _dot(g, v2_r[...]) + c2_r[...]

    sds = jax.ShapeDtypeStruct((e, h), jnp.float32)
    return pl.pallas_call(
        body,
        grid=grid,
        in_specs=[row_spec, row_spec, row_spec,
                  wa_spec, b_spec, wb_spec, b_spec, b_spec, w_spec, b_spec],
        out_specs=[row_spec, row_spec],
        out_shape=(sds, sds),
        compiler_params=pltpu.CompilerParams(
            dimension_semantics=("parallel",)),
    )(src, dst, ea, wa, b1, wb, b2, c1p, pn1["W2"], c2)


def _tc_node(x, s0, s1, c0, c1, pn2):
    n, h = x.shape
    bn = 1000
    assert n % bn == 0
    grid = (n // bn,)

    b1 = pn2["b1"].reshape(1, h)
    b2 = pn2["b2"].reshape(1, h)

    row_spec = pl.BlockSpec((bn, h), lambda i: (i, 0))
    cnt_spec = pl.BlockSpec((bn, h), lambda i: (i, 0))
    w1_spec = pl.BlockSpec((2 * h, h), lambda i: (0, 0))
    w_spec = pl.BlockSpec((h, h), lambda i: (0, 0))
    b_spec = pl.BlockSpec((1, h), lambda i: (0, 0))

    def body(x_r, s0_r, s1_r, c0_r, c1_r, w1_r, b1_r, w2_r, b2_r, out_r):
        cnt = jnp.maximum(c0_r[:, 0:1] + c1_r[:, 0:1], 1.0)
        agg = (s0_r[...] + s1_r[...]) / cnt
        cat = jnp.concatenate([x_r[...], agg], axis=1)
        hh = jnp.maximum(_dot(cat, w1_r[...]) + b1_r[...], 0.0)
        out_r[...] = _dot(hh, w2_r[...]) + b2_r[...]

    return pl.pallas_call(
        body,
        grid=grid,
        in_specs=[row_spec, row_spec, row_spec, cnt_spec, cnt_spec,
                  w1_spec, b_spec, w_spec, b_spec],
        out_specs=pl.BlockSpec((bn, h), lambda i: (i, 0)),
        out_shape=jax.ShapeDtypeStruct((n, h), jnp.float32),
        compiler_params=pltpu.CompilerParams(
            dimension_semantics=("parallel",)),
    )(x, s0, s1, c0, c1, pn2["W1"], b1, pn2["W2"], b2)


def kernel(x, edge_index, edge_attr, params):
    n, h = x.shape
    e = edge_attr.shape[0]
    row = edge_index[0]
    col = edge_index[1]

    zeros_s = jnp.zeros((n, h), jnp.float32)
    cnt_parts = _sc_segsum(edge_attr, col, zeros_s, count_mode=True)
    c0, c1 = cnt_parts[:n], cnt_parts[n:]

    for p in params:
        src, dst = _sc_gather2(x, row, col)
        e_new, m = _tc_edge(src, dst, edge_attr, p["edge"], p["node1"])
        s_parts = _sc_segsum(m, col, zeros_s)
        x = _tc_node(x, s_parts[:n], s_parts[n:], c0, c1, p["node2"])
        edge_attr = e_new
    return (x, edge_attr)


# speedup vs baseline: 3.7501x; 1.3227x over previous
import functools

import jax
import jax.numpy as jnp
from jax import lax
from jax.experimental import pallas as pl
from jax.experimental.pallas import tpu as pltpu
from jax.experimental.pallas import tpu_sc as plsc

NC = 2
NS = 16
NW = NC * NS
LANES = 16
CH = 128

def _mesh():
    return plsc.VectorSubcoreMesh(core_axis_name="c", subcore_axis_name="s")


def _sc_gather2(x, row, col):
    n, h = x.shape
    e = row.shape[0]
    assert e % NW == 0
    epw = e // NW
    n_full, rem = divmod(epw, CH)
    assert epw % 8 == 0 and rem % 8 == 0

    assert n_full % 2 == 0
    npair = n_full // 2

    out_t = jax.ShapeDtypeStruct((e, h), x.dtype)
    scratch = [
        pltpu.VMEM((CH,), jnp.int32), pltpu.VMEM((CH,), jnp.int32),
        pltpu.VMEM((CH,), jnp.int32), pltpu.VMEM((CH,), jnp.int32),
        pltpu.VMEM((CH, h), x.dtype), pltpu.VMEM((CH, h), x.dtype),
        pltpu.VMEM((CH, h), x.dtype), pltpu.VMEM((CH, h), x.dtype),
        pltpu.SemaphoreType.DMA, pltpu.SemaphoreType.DMA,
        pltpu.SemaphoreType.DMA, pltpu.SemaphoreType.DMA,
        pltpu.SemaphoreType.DMA, pltpu.SemaphoreType.DMA,
    ]
    if rem:
        scratch += [
            pltpu.VMEM((rem,), jnp.int32), pltpu.VMEM((rem,), jnp.int32),
            pltpu.VMEM((rem, h), x.dtype), pltpu.VMEM((rem, h), x.dtype),
        ]

    @functools.partial(pl.kernel, out_type=(out_t, out_t), mesh=_mesh(),
                       scratch_types=scratch)
    def k(x_hbm, row_hbm, col_hbm, src_hbm, dst_hbm,
          idxr0, idxr1, idxc0, idxc1, bufr0, bufr1, bufc0, bufc1,
          semi0, semi1, semg0, semg1, semo0, semo1, *tail):
        idxr, idxc = [idxr0, idxr1], [idxc0, idxc1]
        bufr, bufc = [bufr0, bufr1], [bufc0, bufc1]
        semi, semg, semo = [semi0, semi1], [semg0, semg1], [semo0, semo1]
        wid = lax.axis_index("s") * NC + lax.axis_index("c")
        base_w = wid * epw

        def idx_cp(p, b):
            return (pltpu.make_async_copy(row_hbm.at[pl.ds(b, CH)],
                                          idxr[p], semi[p]),
                    pltpu.make_async_copy(col_hbm.at[pl.ds(b, CH)],
                                          idxc[p], semi[p]))

        def gat_cp(p):
            return (pltpu.make_async_copy(x_hbm.at[idxr[p]], bufr[p], semg[p]),
                    pltpu.make_async_copy(x_hbm.at[idxc[p]], bufc[p], semg[p]))

        def out_cp(p, b):
            return (pltpu.make_async_copy(bufr[p], src_hbm.at[pl.ds(b, CH)],
                                          semo[p]),
                    pltpu.make_async_copy(bufc[p], dst_hbm.at[pl.ds(b, CH)],
                                          semo[p]))

        def start2(c):
            c[0].start()
            c[1].start()

        def wait2(c):
            c[0].wait()
            c[1].wait()

        start2(idx_cp(0, base_w))
        start2(idx_cp(1, base_w + CH))
        wait2(idx_cp(0, base_w))
        start2(gat_cp(0))
        wait2(idx_cp(1, base_w + CH))
        start2(gat_cp(1))

        @pl.loop(0, npair)
        def _(j):
            c0 = base_w + (2 * j) * CH
            c1 = c0 + CH
            wait2(gat_cp(0))
            start2(out_cp(0, c0))
            wait2(gat_cp(1))
            start2(out_cp(1, c1))

            @pl.when(j < npair - 1)
            def _():
                start2(idx_cp(0, c0 + 2 * CH))
                start2(idx_cp(1, c1 + 2 * CH))
                wait2(idx_cp(0, c0 + 2 * CH))
                wait2(out_cp(0, c0))
                start2(gat_cp(0))
                wait2(idx_cp(1, c1 + 2 * CH))
                wait2(out_cp(1, c1))
                start2(gat_cp(1))

        blast = base_w + (n_full - 2) * CH
        wait2(out_cp(0, blast))
        wait2(out_cp(1, blast + CH))

        if rem:
            idxrt, idxct, bufrt, bufct = tail
            bt = base_w + n_full * CH
            pltpu.sync_copy(row_hbm.at[pl.ds(bt, rem)], idxrt)
            pltpu.sync_copy(col_hbm.at[pl.ds(bt, rem)], idxct)
            pltpu.sync_copy(x_hbm.at[idxrt], bufrt)
            pltpu.sync_copy(x_hbm.at[idxct], bufct)
            pltpu.sync_copy(bufrt, src_hbm.at[pl.ds(bt, rem)])
            pltpu.sync_copy(bufct, dst_hbm.at[pl.ds(bt, rem)])

    return k(x, row, col)


def _sc_segsum(vals, col, zeros, count_mode=False):
    e, w = vals.shape
    n = zeros.shape[0]
    assert e % NC == 0
    epc = e // NC
    assert epc % NS == 0
    eps = epc // NS
    n_full, rem = divmod(eps, CH)
    assert eps % 8 == 0 and rem % 8 == 0
    rps = (n // NS) & ~7
    rps_last = n - (NS - 1) * rps
    assert rps % 8 == 0 and rps_last % 8 == 0 and rps_last >= rps

    assert n_full % 2 == 0
    npair = n_full // 2

    out_t = jax.ShapeDtypeStruct((NC * n, w), vals.dtype)
    scratch = [
        pltpu.VMEM_SHARED((n, w), vals.dtype),
        pltpu.VMEM((CH,), jnp.int32), pltpu.VMEM((CH,), jnp.int32),
        pltpu.VMEM((CH, w), vals.dtype), pltpu.VMEM((CH, w), vals.dtype),
        pltpu.SemaphoreType.DMA, pltpu.SemaphoreType.DMA,
        pltpu.SemaphoreType.DMA, pltpu.SemaphoreType.DMA,
    ]
    if rem:
        scratch += [pltpu.VMEM((rem,), jnp.int32),
                    pltpu.VMEM((rem, w), vals.dtype)]

    @functools.partial(pl.kernel, out_type=out_t, mesh=_mesh(),
                       scratch_types=scratch)
    def k(vals_hbm, col_hbm, zeros_hbm, out_hbm, acc_sh,
          idx0, idx1, buf0, buf1, semiv0, semiv1, semsc0, semsc1, *tail):
        idxv, bufv = [idx0, idx1], [buf0, buf1]
        semiv, semsc = [semiv0, semiv1], [semsc0, semsc1]
        core = lax.axis_index("c")
        sid = lax.axis_index("s")

        if count_mode:
            @pl.loop(0, CH)
            def _(r):
                @pl.loop(0, w // LANES)
                def _(cc):
                    buf0[r, pl.ds(cc * LANES, LANES)] = jnp.ones(
                        (LANES,), vals.dtype)

        r0 = sid * rps

        @pl.when(sid < NS - 1)
        def _():
            pltpu.sync_copy(zeros_hbm.at[pl.ds(r0, rps)],
                            acc_sh.at[pl.ds(r0, rps)])

        @pl.when(sid == NS - 1)
        def _():
            pltpu.sync_copy(zeros_hbm.at[pl.ds(r0, rps_last)],
                            acc_sh.at[pl.ds(r0, rps_last)])

        plsc.subcore_barrier()

        base = core * epc + sid * eps

        def iv_cp(p, b):
            cs = [pltpu.make_async_copy(col_hbm.at[pl.ds(b, CH)],
                                        idxv[p], semiv[p])]
            if not count_mode:
                cs.append(pltpu.make_async_copy(vals_hbm.at[pl.ds(b, CH)],
                                                bufv[p], semiv[p]))
            return cs

        def sc_src(p):
            return bufv[0] if count_mode else bufv[p]

        def startall(cs):
            for c in cs:
                c.start()

        def waitall(cs):
            for c in cs:
                c.wait()

        def sc_start(p):
            pltpu.async_copy(sc_src(p), acc_sh.at[idxv[p]], semsc[p],
                             add=True)

        def sc_wait(p):
            pltpu.make_async_copy(sc_src(p), acc_sh.at[idxv[p]],
                                  semsc[p]).wait()

        startall(iv_cp(0, base))
        startall(iv_cp(1, base + CH))

        @pl.loop(0, npair)
        def _(j):
            c0 = base + (2 * j) * CH
            c1 = c0 + CH
            waitall(iv_cp(0, c0))
            sc_start(0)
            waitall(iv_cp(1, c1))
            sc_start(1)

            @pl.when(j < npair - 1)
            def _():
                sc_wait(0)
                startall(iv_cp(0, c0 + 2 * CH))
                sc_wait(1)
                startall(iv_cp(1, c1 + 2 * CH))

        sc_wait(0)
        sc_wait(1)

        if rem:
            idx_t, buf_t = tail
            bt = base + n_full * CH
            pltpu.sync_copy(col_hbm.at[pl.ds(bt, rem)], idx_t)
            if count_mode:
                @pl.loop(0, rem)
                def _(r):
                    @pl.loop(0, w // LANES)
                    def _(cc):
                        buf_t[r, pl.ds(cc * LANES, LANES)] = jnp.ones(
                            (LANES,), vals.dtype)
            else:
                pltpu.sync_copy(vals_hbm.at[pl.ds(bt, rem)], buf_t)
            pltpu.sync_copy(buf_t, acc_sh.at[idx_t], add=True)

        plsc.subcore_barrier()

        @pl.when(sid < NS - 1)
        def _():
            pltpu.sync_copy(acc_sh.at[pl.ds(r0, rps)],
                            out_hbm.at[pl.ds(core * n + r0, rps)])

        @pl.when(sid == NS - 1)
        def _():
            pltpu.sync_copy(acc_sh.at[pl.ds(r0, rps_last)],
                            out_hbm.at[pl.ds(core * n + r0, rps_last)])

    return k(vals, col, zeros)


def _dot(a, b):
    return jax.lax.dot_general(a.astype(jnp.bfloat16), b.astype(jnp.bfloat16),
                               (((1,), (0,)), ((), ())),
                               preferred_element_type=jnp.float32)


def _tc_edge(src, dst, ea, pe, pn1):
    e, h = ea.shape
    be = 2560
    assert e % be == 0
    grid = (e // be,)

    w1s, w1d, w1e = pe["W1"][:h], pe["W1"][h:2 * h], pe["W1"][2 * h:]
    v1s, v1e = pn1["W1"][:h], pn1["W1"][h:]
    b1 = pe["b1"].reshape(1, h)
    b2 = pe["b2"].reshape(1, h)
    c2 = pn1["b2"].reshape(1, h)
    z = jnp.zeros((h, h), jnp.float32)
    wa = jnp.concatenate([
        jnp.concatenate([w1s, v1s], axis=1),
        jnp.concatenate([w1d, z], axis=1),
        jnp.concatenate([w1e, z], axis=1)], axis=0)
    w2v = jnp.matmul(pe["W2"], v1e, precision=jax.lax.Precision.HIGHEST)
    wb = jnp.concatenate([pe["W2"], w2v], axis=1)
    c1p = (pn1["b1"]
           + jnp.matmul(b2, v1e,
                        precision=jax.lax.Precision.HIGHEST)).reshape(1, h)

    row_spec = pl.BlockSpec((be, h), lambda i: (i, 0))
    wa_spec = pl.BlockSpec((3 * h, 2 * h), lambda i: (0, 0))
    wb_spec = pl.BlockSpec((h, 2 * h), lambda i: (0, 0))
    w_spec = pl.BlockSpec((h, h), lambda i: (0, 0))
    b_spec = pl.BlockSpec((1, h), lambda i: (0, 0))

    def body(src_r, dst_r, ea_r, wa_r, b1_r, wb_r, b2_r, c1p_r, v2_r, c2_r,
             e_out, m_out):
        cat = jnp.concatenate([src_r[...], dst_r[...], ea_r[...]], axis=1)
        a = _dot(cat, wa_r[...])
        hh = jnp.maximum(a[:, :h] + b1_r[...], 0.0)
        gs = a[:, h:]
        bb = _dot(hh, wb_r[...])
        e2 = bb[:, :h] + b2_r[...]
        e_out[...] = e2
        g = jnp.maximum(gs + bb[:, h:] + c1p_r[...], 0.0)
        m_out[...] = _dot(g, v2_r[...]) + c2_r[...]

    sds = jax.ShapeDtypeStruct((e, h), jnp.float32)
    return pl.pallas_call(
        body,
        grid=grid,
        in_specs=[row_spec, row_spec, row_spec,
                  wa_spec, b_spec, wb_spec, b_spec, b_spec, w_spec, b_spec],
        out_specs=[row_spec, row_spec],
        out_shape=(sds, sds),
        compiler_params=pltpu.CompilerParams(
            dimension_semantics=("parallel",)),
    )(src, dst, ea, wa, b1, wb, b2, c1p, pn1["W2"], c2)


def _tc_node(x, s0, s1, c0, c1, pn2):
    n, h = x.shape
    bn = 1000
    assert n % bn == 0
    grid = (n // bn,)

    b1 = pn2["b1"].reshape(1, h)
    b2 = pn2["b2"].reshape(1, h)

    row_spec = pl.BlockSpec((bn, h), lambda i: (i, 0))
    cnt_spec = pl.BlockSpec((bn, h), lambda i: (i, 0))
    w1_spec = pl.BlockSpec((2 * h, h), lambda i: (0, 0))
    w_spec = pl.BlockSpec((h, h), lambda i: (0, 0))
    b_spec = pl.BlockSpec((1, h), lambda i: (0, 0))

    def body(x_r, s0_r, s1_r, c0_r, c1_r, w1_r, b1_r, w2_r, b2_r, out_r):
        cnt = jnp.maximum(c0_r[:, 0:1] + c1_r[:, 0:1], 1.0)
        agg = (s0_r[...] + s1_r[...]) / cnt
        cat = jnp.concatenate([x_r[...], agg], axis=1)
        hh = jnp.maximum(_dot(cat, w1_r[...]) + b1_r[...], 0.0)
        out_r[...] = _dot(hh, w2_r[...]) + b2_r[...]

    return pl.pallas_call(
        body,
        grid=grid,
        in_specs=[row_spec, row_spec, row_spec, cnt_spec, cnt_spec,
                  w1_spec, b_spec, w_spec, b_spec],
        out_specs=pl.BlockSpec((bn, h), lambda i: (i, 0)),
        out_shape=jax.ShapeDtypeStruct((n, h), jnp.float32),
        compiler_params=pltpu.CompilerParams(
            dimension_semantics=("parallel",)),
    )(x, s0, s1, c0, c1, pn2["W1"], b1, pn2["W2"], b2)


def kernel(x, edge_index, edge_attr, params):
    n, h = x.shape
    e = edge_attr.shape[0]
    row = edge_index[0]
    col = edge_index[1]

    zeros_s = jnp.zeros((n, h), jnp.float32)
    cnt_parts = _sc_segsum(edge_attr, col, zeros_s, count_mode=True)
    c0, c1 = cnt_parts[:n], cnt_parts[n:]

    for p in params:
        src, dst = _sc_gather2(x, row, col)
        e_new, m = _tc_edge(src, dst, edge_attr, p["edge"], p["node1"])
        s_parts = _sc_segsum(m, col, zeros_s)
        x = _tc_node(x, s_parts[:n], s_parts[n:], c0, c1, p["node2"])
        edge_attr = e_new
    return (x, edge_attr)
